# baseline (speedup 1.0000x reference)
import jax
import jax.numpy as jnp
from jax.experimental import pallas as pl
from jax.experimental.pallas import tpu as pltpu

_E = 8
_C = 512
_BT = 256


def _router_kernel(x_ref, w_ref, b_ref, out1_ref, out2_ref, cnt_ref):
    i = pl.program_id(0)

    @pl.when(i == 0)
    def _():
        cnt_ref[...] = jnp.zeros_like(cnt_ref)

    x = x_ref[...]
    w = w_ref[...]
    logits = jnp.dot(x, w, preferred_element_type=jnp.float32) + b_ref[...]
    maxv = jnp.max(logits, axis=1, keepdims=True)
    denom = jnp.sum(jnp.exp(logits - maxv), axis=1, keepdims=True)
    gate = 1.0 / denom

    lane = jax.lax.broadcasted_iota(jnp.int32, logits.shape, 1)
    eidx = jnp.min(jnp.where(logits == maxv, lane, _E), axis=1,
                   keepdims=True)
    m = (lane == eidx).astype(jnp.float32)

    bt = m.shape[0]
    row = jax.lax.broadcasted_iota(jnp.int32, (bt, bt), 0)
    col = jax.lax.broadcasted_iota(jnp.int32, (bt, bt), 1)
    tri = (col <= row).astype(jnp.float32)
    cs = jnp.dot(tri, m, preferred_element_type=jnp.float32)
    pos = cs + cnt_ref[...]
    cnt_ref[...] += jnp.sum(m, axis=0, keepdims=True)

    p = jnp.sum(pos * m, axis=1, keepdims=True)
    kept = (p < float(_C)).astype(jnp.float32)
    gate_eff = gate * kept

    e8 = jnp.repeat(eidx, _E, axis=0)
    p8 = jnp.repeat(p.astype(jnp.int32), _E, axis=0)
    g8 = jnp.repeat(gate_eff, _E, axis=0)
    r = jax.lax.broadcasted_iota(jnp.int32, (_E * bt, 1), 0)
    erow = jax.lax.rem(r, _E)
    out_col = jax.lax.broadcasted_iota(jnp.int32, (_E * bt, _C), 1)
    block = jnp.where((erow == e8) & (out_col == p8), g8, 0.0)
    out1_ref[...] = block
    out2_ref[...] = block


def kernel(inputs, W, b):
    t, d = inputs.shape
    e = W.shape[1]
    flat = jax.ShapeDtypeStruct((t * e, _C), jnp.float32)
    out1, out2 = pl.pallas_call(
        _router_kernel,
        grid=(t // _BT,),
        in_specs=[
            pl.BlockSpec((_BT, d), lambda i: (i, 0)),
            pl.BlockSpec((d, e), lambda i: (0, 0)),
            pl.BlockSpec((1, e), lambda i: (0, 0)),
        ],
        out_specs=[pl.BlockSpec((_E * _BT, _C), lambda i: (i, 0))] * 2,
        out_shape=[flat, flat],
        scratch_shapes=[pltpu.VMEM((1, e), jnp.float32)],
    )(inputs, W, b.reshape(1, e))
    return out1.reshape(t, e, _C), out2.reshape(t, e, _C)


# speedup vs baseline: 1.0379x; 1.0379x over previous
import jax
import jax.numpy as jnp
from jax.experimental import pallas as pl
from jax.experimental.pallas import tpu as pltpu

_E = 8
_C = 512
_BT = 512


def _router_kernel(x_ref, w_ref, b_ref, out1_ref, out2_ref, cnt_ref):
    i = pl.program_id(0)

    @pl.when(i == 0)
    def _():
        cnt_ref[...] = jnp.zeros_like(cnt_ref)

    x = x_ref[...]
    w = w_ref[...]
    logits = jnp.dot(x, w, preferred_element_type=jnp.float32) + b_ref[...]
    maxv = jnp.max(logits, axis=1, keepdims=True)
    denom = jnp.sum(jnp.exp(logits - maxv), axis=1, keepdims=True)
    gate = 1.0 / denom

    lane = jax.lax.broadcasted_iota(jnp.int32, logits.shape, 1)
    eidx = jnp.min(jnp.where(logits == maxv, lane, _E), axis=1,
                   keepdims=True)
    m = (lane == eidx).astype(jnp.float32)

    bt = m.shape[0]
    row = jax.lax.broadcasted_iota(jnp.int32, (bt, bt), 0)
    col = jax.lax.broadcasted_iota(jnp.int32, (bt, bt), 1)
    tri = (col <= row).astype(jnp.float32)
    cs = jnp.dot(tri, m, preferred_element_type=jnp.float32)
    pos = cs + cnt_ref[...]
    cnt_ref[...] += jnp.sum(m, axis=0, keepdims=True)

    p = jnp.sum(pos * m, axis=1, keepdims=True)
    kept = (p < float(_C)).astype(jnp.float32)
    gate_eff = gate * kept

    e8 = jnp.repeat(eidx, _E, axis=0)
    p8 = jnp.repeat(p.astype(jnp.int32), _E, axis=0)
    g8 = jnp.repeat(gate_eff, _E, axis=0)
    r = jax.lax.broadcasted_iota(jnp.int32, (_E * bt, 1), 0)
    erow = jax.lax.rem(r, _E)
    out_col = jax.lax.broadcasted_iota(jnp.int32, (_E * bt, _C), 1)
    block = jnp.where((erow == e8) & (out_col == p8), g8, 0.0)
    out1_ref[...] = block
    out2_ref[...] = block


def kernel(inputs, W, b):
    t, d = inputs.shape
    e = W.shape[1]
    flat = jax.ShapeDtypeStruct((t * e, _C), jnp.float32)
    out1, out2 = pl.pallas_call(
        _router_kernel,
        grid=(t // _BT,),
        in_specs=[
            pl.BlockSpec((_BT, d), lambda i: (i, 0)),
            pl.BlockSpec((d, e), lambda i: (0, 0)),
            pl.BlockSpec((1, e), lambda i: (0, 0)),
        ],
        out_specs=[pl.BlockSpec((_E * _BT, _C), lambda i: (i, 0))] * 2,
        out_shape=[flat, flat],
        scratch_shapes=[pltpu.VMEM((1, e), jnp.float32)],
    )(inputs, W, b.reshape(1, e))
    return out1.reshape(t, e, _C), out2.reshape(t, e, _C)
